# Initial kernel scaffold; baseline (speedup 1.0000x reference)
import jax
import jax.numpy as jnp
from jax.experimental import pallas as pl


def kernel(x, edge_index, batch, W0, b0, W1a, b1a, W1b, b1b, W2a, b2a, W2b, b2b, W3a, b3a, W3b, b3b, Wout, bout):
    raise NotImplementedError("write your pallas kernel here")


# speedup vs baseline: 6.4067x; 6.4067x over previous
import functools

import jax
import jax.numpy as jnp
from jax import lax
from jax.experimental import pallas as pl
from jax.experimental.pallas import tpu as pltpu
from jax.experimental.pallas import tpu_sc as plsc

N_NODES = 10000
N_PAD = 10240
D = 128
E = 320000
N_GRAPHS = 128
G_PAD = 144

NC = 2
NS = 16
NW = NC * NS
EW = E // NW
CH = 80
NCHUNK = EW // CH
ROWS_PER_TILE = N_PAD // NS
NODES_W = N_PAD // NW
PCH = NODES_W // CH

_sc_mesh = lambda: plsc.VectorSubcoreMesh(
    core_axis_name="c", subcore_axis_name="s", num_cores=NC, num_subcores=NS)


def _agg_body(h_hbm, src_hbm, dst_hbm, zeros_hbm,
              out0_hbm, out1_hbm,
              src_v, dst_v, rows_v, sem, agg_s):
    cid = lax.axis_index("c")
    sid = lax.axis_index("s")
    wid = cid * NS + sid
    pltpu.sync_copy(zeros_hbm, agg_s.at[pl.ds(sid * ROWS_PER_TILE, ROWS_PER_TILE)])
    pltpu.sync_copy(src_hbm.at[pl.ds(wid * EW, EW)], src_v)
    pltpu.sync_copy(dst_hbm.at[wid], dst_v)
    plsc.subcore_barrier()

    def body(c, carry):
        idx = src_v.at[pl.ds(c * CH, CH)]
        pltpu.async_copy(h_hbm.at[idx], rows_v, sem).wait()
        pltpu.sync_copy(rows_v, agg_s.at[dst_v.at[c]], add=True)
        return carry

    lax.fori_loop(0, NCHUNK, body, 0)
    plsc.subcore_barrier()

    @pl.when(cid == 0)
    def _():
        pltpu.sync_copy(agg_s.at[pl.ds(sid * ROWS_PER_TILE, ROWS_PER_TILE)],
                        out0_hbm.at[pl.ds(sid * ROWS_PER_TILE, ROWS_PER_TILE)])

    @pl.when(cid == 1)
    def _():
        pltpu.sync_copy(agg_s.at[pl.ds(sid * ROWS_PER_TILE, ROWS_PER_TILE)],
                        out1_hbm.at[pl.ds(sid * ROWS_PER_TILE, ROWS_PER_TILE)])


_agg_call = pl.kernel(
    _agg_body,
    out_type=(jax.ShapeDtypeStruct((N_PAD, D), jnp.float32),
              jax.ShapeDtypeStruct((N_PAD, D), jnp.float32)),
    mesh=_sc_mesh(),
    scratch_types=[
        pltpu.VMEM((EW,), jnp.int32),
        pltpu.VMEM((NCHUNK, CH), jnp.int32),
        pltpu.VMEM((CH, D), jnp.float32),
        pltpu.SemaphoreType.DMA,
        pltpu.VMEM_SHARED((N_PAD, D), jnp.float32),
    ],
)


def _pool_body(h_hbm, batch_hbm, zg_hbm, ones_hbm,
               p0_hbm, p1_hbm, c0_hbm, c1_hbm,
               bat_v, rows_v, ones_v, pool_s, cnt_s):
    cid = lax.axis_index("c")
    sid = lax.axis_index("s")
    wid = cid * NS + sid

    @pl.when(sid == 0)
    def _():
        pltpu.sync_copy(zg_hbm, pool_s)
        pltpu.sync_copy(zg_hbm, cnt_s)

    pltpu.sync_copy(batch_hbm.at[wid], bat_v)
    pltpu.sync_copy(ones_hbm, ones_v)
    plsc.subcore_barrier()

    def body(k, carry):
        pltpu.sync_copy(h_hbm.at[pl.ds(wid * NODES_W + k * CH, CH)], rows_v)
        pltpu.sync_copy(rows_v, pool_s.at[bat_v.at[k]], add=True)
        pltpu.sync_copy(ones_v, cnt_s.at[bat_v.at[k]], add=True)
        return carry

    lax.fori_loop(0, PCH, body, 0)
    plsc.subcore_barrier()

    @pl.when(sid == 0)
    def _():
        @pl.when(cid == 0)
        def _():
            pltpu.sync_copy(pool_s, p0_hbm)
            pltpu.sync_copy(cnt_s, c0_hbm)

        @pl.when(cid == 1)
        def _():
            pltpu.sync_copy(pool_s, p1_hbm)
            pltpu.sync_copy(cnt_s, c1_hbm)


_pool_call = pl.kernel(
    _pool_body,
    out_type=(jax.ShapeDtypeStruct((G_PAD, D), jnp.float32),
              jax.ShapeDtypeStruct((G_PAD, D), jnp.float32),
              jax.ShapeDtypeStruct((G_PAD, D), jnp.float32),
              jax.ShapeDtypeStruct((G_PAD, D), jnp.float32)),
    mesh=_sc_mesh(),
    scratch_types=[
        pltpu.VMEM((PCH, CH), jnp.int32),
        pltpu.VMEM((CH, D), jnp.float32),
        pltpu.VMEM((CH, D), jnp.float32),
        pltpu.VMEM_SHARED((G_PAD, D), jnp.float32),
        pltpu.VMEM_SHARED((G_PAD, D), jnp.float32),
    ],
)


_BLK = 1280
_NBLK = N_PAD // _BLK


def _lin_relu_tc(x_ref, w_ref, b_ref, o_ref):
    o_ref[...] = jnp.maximum(
        jnp.dot(x_ref[...], w_ref[...], preferred_element_type=jnp.float32)
        + b_ref[...], 0.0)


def _mlp_tc(x_ref, a0_ref, a1_ref, wa_ref, ba_ref, wb_ref, bb_ref, o_ref):
    t = x_ref[...] + a0_ref[...] + a1_ref[...]
    t = jnp.maximum(
        jnp.dot(t, wa_ref[...], preferred_element_type=jnp.float32)
        + ba_ref[...], 0.0)
    t = jnp.dot(t, wb_ref[...], preferred_element_type=jnp.float32) + bb_ref[...]
    o_ref[...] = jnp.maximum(t, 0.0)


def _final_tc(p0_ref, p1_ref, c0_ref, c1_ref, w_ref, b_ref, o_ref):
    sums = p0_ref[:N_GRAPHS, :] + p1_ref[:N_GRAPHS, :]
    cnts = c0_ref[:N_GRAPHS, :] + c1_ref[:N_GRAPHS, :]
    pooled = sums / jnp.maximum(cnts, 1.0)
    o_ref[...] = (jnp.dot(pooled, w_ref[...], preferred_element_type=jnp.float32)
                  + b_ref[...])


def _row_spec(shape):
    return pl.BlockSpec(shape, lambda i: (i, 0))


def _const_spec(shape):
    return pl.BlockSpec(shape, lambda i: (0, 0))


_lin_relu_call = pl.pallas_call(
    _lin_relu_tc,
    grid=(_NBLK,),
    in_specs=[_row_spec((_BLK, D)), _const_spec((D, D)), _const_spec((1, D))],
    out_specs=_row_spec((_BLK, D)),
    out_shape=jax.ShapeDtypeStruct((N_PAD, D), jnp.float32),
)

_mlp_call = pl.pallas_call(
    _mlp_tc,
    grid=(_NBLK,),
    in_specs=[_row_spec((_BLK, D)), _row_spec((_BLK, D)), _row_spec((_BLK, D)),
              _const_spec((D, D)), _const_spec((1, D)),
              _const_spec((D, D)), _const_spec((1, D))],
    out_specs=_row_spec((_BLK, D)),
    out_shape=jax.ShapeDtypeStruct((N_PAD, D), jnp.float32),
)

_final_call = pl.pallas_call(
    _final_tc,
    in_specs=[pl.BlockSpec((G_PAD, D), lambda: (0, 0))] * 4
             + [pl.BlockSpec((D, D), lambda: (0, 0)),
                pl.BlockSpec((1, D), lambda: (0, 0))],
    out_specs=pl.BlockSpec((N_GRAPHS, D), lambda: (0, 0)),
    out_shape=jax.ShapeDtypeStruct((N_GRAPHS, D), jnp.float32),
)


def kernel(x, edge_index, batch, W0, b0, W1a, b1a, W1b, b1b, W2a, b2a,
           W2b, b2b, W3a, b3a, W3b, b3b, Wout, bout):
    x_pad = jnp.pad(x, ((0, N_PAD - N_NODES), (0, 0)))
    src = edge_index[0]
    dst = edge_index[1].reshape(NW, NCHUNK, CH)
    batch_pad = jnp.pad(batch, (0, N_PAD - N_NODES),
                        constant_values=N_GRAPHS).reshape(NW, PCH, CH)
    zeros_rows = jnp.zeros((ROWS_PER_TILE, D), jnp.float32)
    zeros_g = jnp.zeros((G_PAD, D), jnp.float32)
    ones_rows = jnp.ones((CH, D), jnp.float32)
    Wout_pad = jnp.pad(Wout, ((0, 0), (0, D - Wout.shape[1])))
    bout_pad = jnp.pad(bout, (0, D - bout.shape[0])).reshape(1, D)

    h = _lin_relu_call(x_pad, W0, b0.reshape(1, D))
    for Wa, ba, Wb, bb in ((W1a, b1a, W1b, b1b),
                           (W2a, b2a, W2b, b2b),
                           (W3a, b3a, W3b, b3b)):
        a0, a1 = _agg_call(h, src, dst, zeros_rows)
        h = _mlp_call(h, a0, a1, Wa, ba.reshape(1, D), Wb, bb.reshape(1, D))

    p0, p1, c0, c1 = _pool_call(h, batch_pad, zeros_g, ones_rows)
    out = _final_call(p0, p1, c0, c1, Wout_pad, bout_pad)
    return out[:, :Wout.shape[1]]
